# baseline (speedup 1.0000x reference)
import functools

import jax
import jax.numpy as jnp
from jax import lax
from jax.experimental import pallas as pl
from jax.experimental.pallas import tpu as pltpu
from jax.experimental.pallas import tpu_sc as plsc

HASH_N = 20480
EMB = 128
DM = 2048
P1C = 31337

BATCH = 4
SEQ = 4096
NTOK = BATCH * SEQ
NWORK = 32

SLICES = (2048, 2048, 4096, 8192)
ROWS_PER_DMA = 128

CT = 512
NBUF = 6


def _sc_gather_kernel(ids_hbm, table_hbm, emb_hbm, ids_v, h_v, rows_v, sem,
                      *, off, chunk):
    groups = chunk // 16
    nblk = (chunk + ROWS_PER_DMA - 1) // ROWS_PER_DMA

    wid = lax.axis_index("s") * 2 + lax.axis_index("c")
    base = off + wid * chunk

    @pl.when(base % SEQ == 0)
    def _():
        ids_v[pl.ds(0, 16)] = jnp.zeros((16,), jnp.int32)
        pltpu.sync_copy(ids_hbm.at[pl.ds(base, chunk)], ids_v.at[pl.ds(8, chunk)])

    @pl.when(base % SEQ != 0)
    def _():
        pltpu.sync_copy(ids_hbm.at[pl.ds(base - 8, chunk + 8)], ids_v)

    for g in range(groups):
        cur = ids_v[pl.ds(8 + g * 16, 16)]
        prev = ids_v[pl.ds(7 + g * 16, 16)]
        x = (prev * P1C) ^ cur
        r = lax.rem(x, HASH_N)
        h = jnp.where(r < 0, r + HASH_N, r)
        h_v[g // (ROWS_PER_DMA // 16), pl.ds((g % (ROWS_PER_DMA // 16)) * 16, 16)] = h
    for g in range(groups, nblk * (ROWS_PER_DMA // 16)):
        h_v[g // (ROWS_PER_DMA // 16), pl.ds((g % (ROWS_PER_DMA // 16)) * 16, 16)] = (
            jnp.zeros((16,), jnp.int32))

    cps = [
        pltpu.async_copy(
            table_hbm.at[h_v.at[j]],
            rows_v.at[pl.ds(j * ROWS_PER_DMA, ROWS_PER_DMA)],
            sem,
        )
        for j in range(nblk)
    ]
    for cp in cps:
        cp.wait()
    pltpu.sync_copy(rows_v.at[pl.ds(0, chunk)], emb_hbm.at[pl.ds(base - off, chunk)])


def _sc_gather(ids_flat, table, off, tok_s):
    chunk = tok_s // NWORK
    nblk = (chunk + ROWS_PER_DMA - 1) // ROWS_PER_DMA
    mesh = plsc.VectorSubcoreMesh(core_axis_name="c", subcore_axis_name="s")
    body = functools.partial(_sc_gather_kernel, off=off, chunk=chunk)
    fn = functools.partial(
        pl.kernel,
        mesh=mesh,
        out_type=jax.ShapeDtypeStruct((tok_s, EMB), jnp.float32),
        scratch_types=[
            pltpu.VMEM((chunk + 8,), jnp.int32),
            pltpu.VMEM((nblk, ROWS_PER_DMA), jnp.int32),
            pltpu.VMEM((nblk * ROWS_PER_DMA, EMB), jnp.float32),
            pltpu.SemaphoreType.DMA,
        ],
    )(body)
    return fn(ids_flat, table)


def _mm_slice_body(x_ref, w_ref, *rest, off, tok_s):
    o_hbm, ob, sems = rest[-3], rest[-2], rest[-1]
    nc = tok_s // CT
    w = w_ref[...].astype(jnp.bfloat16)
    cps = [None] * NBUF
    for j in range(nc):
        b = j % NBUF
        if cps[b] is not None:
            cps[b].wait()
        xj = x_ref[pl.ds(j * CT, CT), :].astype(jnp.bfloat16)
        ob[b] = lax.dot_general(
            xj, w,
            dimension_numbers=(((1,), (1,)), ((), ())),
            preferred_element_type=jnp.float32,
        )
        cps[b] = pltpu.make_async_copy(
            ob.at[b], o_hbm.at[pl.ds(off + j * CT, CT), :], sems.at[b]
        )
        cps[b].start()
    for b in range(NBUF):
        if cps[b] is not None:
            cps[b].wait()


def _project_slice(emb_s, proj_w, out, off, tok_s):
    body = functools.partial(_mm_slice_body, off=off, tok_s=tok_s)
    vmem = pl.BlockSpec(memory_space=pltpu.MemorySpace.VMEM)
    hbm = pl.BlockSpec(memory_space=pltpu.MemorySpace.HBM)
    scratch = [
        pltpu.VMEM((NBUF, CT, DM), jnp.float32),
        pltpu.SemaphoreType.DMA((NBUF,)),
    ]
    if out is None:
        return pl.pallas_call(
            body,
            in_specs=[vmem, vmem],
            out_specs=hbm,
            out_shape=jax.ShapeDtypeStruct((NTOK, DM), jnp.float32),
            scratch_shapes=scratch,
        )(emb_s, proj_w)
    return pl.pallas_call(
        body,
        in_specs=[vmem, vmem, hbm],
        out_specs=hbm,
        out_shape=jax.ShapeDtypeStruct((NTOK, DM), jnp.float32),
        scratch_shapes=scratch,
        input_output_aliases={2: 0},
    )(emb_s, proj_w, out)


@jax.jit
def kernel(input_ids, bigram_emb, proj_w):
    ids_flat = input_ids.reshape(-1)
    offs = []
    o = 0
    for t in SLICES:
        offs.append(o)
        o += t
    embs = [
        _sc_gather(ids_flat, bigram_emb, off, tok_s)
        for off, tok_s in zip(offs, SLICES)
    ]
    out = None
    for emb_s, off, tok_s in zip(embs, offs, SLICES):
        out = _project_slice(emb_s, proj_w, out, off, tok_s)
    return out.reshape(BATCH, SEQ, DM)


# speedup vs baseline: 2.8708x; 2.8708x over previous
import functools

import jax
import jax.numpy as jnp
from jax import lax
from jax.experimental import pallas as pl
from jax.experimental.pallas import tpu as pltpu
from jax.experimental.pallas import tpu_sc as plsc

HASH_N = 20480
EMB = 128
DM = 2048
P1C = 31337

BATCH = 4
SEQ = 4096
NTOK = BATCH * SEQ
NWORK = 32

SLICES = (2048, 2048, 4096, 8192)
ROWS_PER_DMA = 128

CT = 512
NBUF = 6


def _sc_gather_kernel(ids_hbm, table_hbm, emb_hbm, ids_v, h_v, rows_v, sem,
                      *, off, chunk):
    groups = chunk // 16
    blk = min(chunk, ROWS_PER_DMA)
    nblk = chunk // blk

    wid = lax.axis_index("s") * 2 + lax.axis_index("c")
    base = off + wid * chunk

    @pl.when(base % SEQ == 0)
    def _():
        ids_v[pl.ds(0, 16)] = jnp.zeros((16,), jnp.int32)
        pltpu.sync_copy(ids_hbm.at[pl.ds(base, chunk)], ids_v.at[pl.ds(8, chunk)])

    @pl.when(base % SEQ != 0)
    def _():
        pltpu.sync_copy(ids_hbm.at[pl.ds(base - 8, chunk + 8)], ids_v)

    for g in range(groups):
        cur = ids_v[pl.ds(8 + g * 16, 16)]
        prev = ids_v[pl.ds(7 + g * 16, 16)]
        x = (prev * P1C) ^ cur
        r = lax.rem(x, HASH_N)
        h = jnp.where(r < 0, r + HASH_N, r)
        h_v[g // (blk // 16), pl.ds((g % (blk // 16)) * 16, 16)] = h

    cps = [
        pltpu.async_copy(
            table_hbm.at[h_v.at[j]],
            rows_v.at[pl.ds(j * blk, blk)],
            sem,
        )
        for j in range(nblk)
    ]
    for cp in cps:
        cp.wait()
    pltpu.sync_copy(rows_v, emb_hbm.at[pl.ds(base - off, chunk)])


def _sc_gather(ids_flat, table, off, tok_s):
    chunk = tok_s // NWORK
    blk = min(chunk, ROWS_PER_DMA)
    nblk = chunk // blk
    mesh = plsc.VectorSubcoreMesh(core_axis_name="c", subcore_axis_name="s")
    body = functools.partial(_sc_gather_kernel, off=off, chunk=chunk)
    fn = functools.partial(
        pl.kernel,
        mesh=mesh,
        out_type=jax.ShapeDtypeStruct((tok_s, EMB), jnp.float32),
        scratch_types=[
            pltpu.VMEM((chunk + 8,), jnp.int32),
            pltpu.VMEM((nblk, blk), jnp.int32),
            pltpu.VMEM((chunk, EMB), jnp.float32),
            pltpu.SemaphoreType.DMA,
        ],
    )(body)
    return fn(ids_flat, table)


def _mm_slice_body(x_ref, w_ref, *rest, off, tok_s):
    o_hbm, ob, sems = rest[-3], rest[-2], rest[-1]
    nc = tok_s // CT
    w = w_ref[...].astype(jnp.bfloat16)
    cps = [None] * NBUF
    for j in range(nc):
        b = j % NBUF
        if cps[b] is not None:
            cps[b].wait()
        xj = x_ref[pl.ds(j * CT, CT), :].astype(jnp.bfloat16)
        ob[b] = lax.dot_general(
            xj, w,
            dimension_numbers=(((1,), (1,)), ((), ())),
            preferred_element_type=jnp.float32,
        )
        cps[b] = pltpu.make_async_copy(
            ob.at[b], o_hbm.at[pl.ds(off + j * CT, CT), :], sems.at[b]
        )
        cps[b].start()
    for b in range(NBUF):
        if cps[b] is not None:
            cps[b].wait()


def _project_slice(emb_s, proj_w, out, off, tok_s):
    body = functools.partial(_mm_slice_body, off=off, tok_s=tok_s)
    vmem = pl.BlockSpec(memory_space=pltpu.MemorySpace.VMEM)
    hbm = pl.BlockSpec(memory_space=pltpu.MemorySpace.HBM)
    scratch = [
        pltpu.VMEM((NBUF, CT, DM), jnp.float32),
        pltpu.SemaphoreType.DMA((NBUF,)),
    ]
    if out is None:
        return pl.pallas_call(
            body,
            in_specs=[vmem, vmem],
            out_specs=hbm,
            out_shape=jax.ShapeDtypeStruct((NTOK, DM), jnp.float32),
            scratch_shapes=scratch,
        )(emb_s, proj_w)
    return pl.pallas_call(
        body,
        in_specs=[vmem, vmem, hbm],
        out_specs=hbm,
        out_shape=jax.ShapeDtypeStruct((NTOK, DM), jnp.float32),
        scratch_shapes=scratch,
        input_output_aliases={2: 0},
    )(emb_s, proj_w, out)


@jax.jit
def kernel(input_ids, bigram_emb, proj_w):
    ids_flat = input_ids.reshape(-1)
    offs = []
    o = 0
    for t in SLICES:
        offs.append(o)
        o += t
    embs = [
        _sc_gather(ids_flat, bigram_emb, off, tok_s)
        for off, tok_s in zip(offs, SLICES)
    ]
    out = None
    for emb_s, off, tok_s in zip(embs, offs, SLICES):
        out = _project_slice(emb_s, proj_w, out, off, tok_s)
    return out.reshape(BATCH, SEQ, DM)


# speedup vs baseline: 3.0600x; 1.0659x over previous
import functools

import jax
import jax.numpy as jnp
from jax import lax
from jax.experimental import pallas as pl
from jax.experimental.pallas import tpu as pltpu
from jax.experimental.pallas import tpu_sc as plsc

HASH_N = 20480
EMB = 128
DM = 2048
P1C = 31337

BATCH = 4
SEQ = 4096
NTOK = BATCH * SEQ
NWORK = 32

SLICES = (4096, 12288)
ROWS_PER_DMA = 128

CT = 512
NBUF = 6


def _sc_gather_kernel(ids_hbm, table_hbm, emb_hbm, ids_v, h_v, rows_v, sem,
                      *, off, chunk):
    groups = chunk // 16
    blk = min(chunk, ROWS_PER_DMA)
    nblk = chunk // blk

    wid = lax.axis_index("s") * 2 + lax.axis_index("c")
    base = off + wid * chunk

    @pl.when(base % SEQ == 0)
    def _():
        ids_v[pl.ds(0, 16)] = jnp.zeros((16,), jnp.int32)
        pltpu.sync_copy(ids_hbm.at[pl.ds(base, chunk)], ids_v.at[pl.ds(8, chunk)])

    @pl.when(base % SEQ != 0)
    def _():
        pltpu.sync_copy(ids_hbm.at[pl.ds(base - 8, chunk + 8)], ids_v)

    for g in range(groups):
        cur = ids_v[pl.ds(8 + g * 16, 16)]
        prev = ids_v[pl.ds(7 + g * 16, 16)]
        x = (prev * P1C) ^ cur
        r = lax.rem(x, HASH_N)
        h = jnp.where(r < 0, r + HASH_N, r)
        h_v[g // (blk // 16), pl.ds((g % (blk // 16)) * 16, 16)] = h

    cps = [
        pltpu.async_copy(
            table_hbm.at[h_v.at[j]],
            rows_v.at[pl.ds(j * blk, blk)],
            sem,
        )
        for j in range(nblk)
    ]
    for cp in cps:
        cp.wait()
    pltpu.sync_copy(rows_v, emb_hbm.at[pl.ds(base - off, chunk)])


def _sc_gather(ids_flat, table, off, tok_s):
    chunk = tok_s // NWORK
    blk = min(chunk, ROWS_PER_DMA)
    nblk = chunk // blk
    mesh = plsc.VectorSubcoreMesh(core_axis_name="c", subcore_axis_name="s")
    body = functools.partial(_sc_gather_kernel, off=off, chunk=chunk)
    fn = functools.partial(
        pl.kernel,
        mesh=mesh,
        out_type=jax.ShapeDtypeStruct((tok_s, EMB), jnp.float32),
        scratch_types=[
            pltpu.VMEM((chunk + 8,), jnp.int32),
            pltpu.VMEM((nblk, blk), jnp.int32),
            pltpu.VMEM((chunk, EMB), jnp.float32),
            pltpu.SemaphoreType.DMA,
        ],
    )(body)
    return fn(ids_flat, table)


def _mm_slice_body(x_ref, w_ref, *rest, off, tok_s):
    o_hbm, ob, sems = rest[-3], rest[-2], rest[-1]
    nc = tok_s // CT
    w = w_ref[...].astype(jnp.bfloat16)
    cps = [None] * NBUF
    for j in range(nc):
        b = j % NBUF
        if cps[b] is not None:
            cps[b].wait()
        xj = x_ref[pl.ds(j * CT, CT), :].astype(jnp.bfloat16)
        ob[b] = lax.dot_general(
            xj, w,
            dimension_numbers=(((1,), (1,)), ((), ())),
            preferred_element_type=jnp.float32,
        )
        cps[b] = pltpu.make_async_copy(
            ob.at[b], o_hbm.at[pl.ds(off + j * CT, CT), :], sems.at[b]
        )
        cps[b].start()
    for b in range(NBUF):
        if cps[b] is not None:
            cps[b].wait()


def _project_slice(emb_s, proj_w, out, off, tok_s):
    body = functools.partial(_mm_slice_body, off=off, tok_s=tok_s)
    vmem = pl.BlockSpec(memory_space=pltpu.MemorySpace.VMEM)
    hbm = pl.BlockSpec(memory_space=pltpu.MemorySpace.HBM)
    scratch = [
        pltpu.VMEM((NBUF, CT, DM), jnp.float32),
        pltpu.SemaphoreType.DMA((NBUF,)),
    ]
    if out is None:
        return pl.pallas_call(
            body,
            in_specs=[vmem, vmem],
            out_specs=hbm,
            out_shape=jax.ShapeDtypeStruct((NTOK, DM), jnp.float32),
            scratch_shapes=scratch,
        )(emb_s, proj_w)
    return pl.pallas_call(
        body,
        in_specs=[vmem, vmem, hbm],
        out_specs=hbm,
        out_shape=jax.ShapeDtypeStruct((NTOK, DM), jnp.float32),
        scratch_shapes=scratch,
        input_output_aliases={2: 0},
    )(emb_s, proj_w, out)


@jax.jit
def kernel(input_ids, bigram_emb, proj_w):
    ids_flat = input_ids.reshape(-1)
    offs = []
    o = 0
    for t in SLICES:
        offs.append(o)
        o += t
    embs = [
        _sc_gather(ids_flat, bigram_emb, off, tok_s)
        for off, tok_s in zip(offs, SLICES)
    ]
    out = None
    for emb_s, off, tok_s in zip(embs, offs, SLICES):
        out = _project_slice(emb_s, proj_w, out, off, tok_s)
    return out.reshape(BATCH, SEQ, DM)


# speedup vs baseline: 3.0618x; 1.0006x over previous
import functools

import jax
import jax.numpy as jnp
from jax import lax
from jax.experimental import pallas as pl
from jax.experimental.pallas import tpu as pltpu
from jax.experimental.pallas import tpu_sc as plsc

HASH_N = 20480
EMB = 128
DM = 2048
P1C = 31337

BATCH = 4
SEQ = 4096
NTOK = BATCH * SEQ
NWORK = 32

SLICES = (4096, 12288)
ROWS_PER_DMA = 128

CT = 512
NBUF = 6


def _sc_gather_kernel(ids_hbm, table_hbm, emb_hbm, ids_v, h_v, rows_v, sem,
                      *, off, chunk):
    groups = chunk // 16
    blk = min(chunk, ROWS_PER_DMA)
    nblk = chunk // blk

    wid = lax.axis_index("s") * 2 + lax.axis_index("c")
    base = off + wid * chunk

    @pl.when(base % SEQ == 0)
    def _():
        ids_v[pl.ds(0, 16)] = jnp.zeros((16,), jnp.int32)
        pltpu.sync_copy(ids_hbm.at[pl.ds(base, chunk)], ids_v.at[pl.ds(8, chunk)])

    @pl.when(base % SEQ != 0)
    def _():
        pltpu.sync_copy(ids_hbm.at[pl.ds(base - 8, chunk + 8)], ids_v)

    lane = lax.iota(jnp.int32, 16)
    for g in range(groups):
        cur = ids_v[pl.ds(8 + g * 16, 16)]
        prev = ids_v[pl.ds(7 + g * 16, 16)]
        posm = (base + g * 16 + lane) & (SEQ - 1)
        prev = jnp.where(posm == 0, 0, prev)
        x = (prev * P1C) ^ cur
        r = lax.rem(x, HASH_N)
        h = jnp.where(r < 0, r + HASH_N, r)
        h_v[g // (blk // 16), pl.ds((g % (blk // 16)) * 16, 16)] = h

    cps = [
        pltpu.async_copy(
            table_hbm.at[h_v.at[j]],
            rows_v.at[pl.ds(j * blk, blk)],
            sem,
        )
        for j in range(nblk)
    ]
    for cp in cps:
        cp.wait()
    pltpu.sync_copy(rows_v, emb_hbm.at[pl.ds(base - off, chunk)])


def _sc_gather(ids_flat, table, off, tok_s):
    chunk = tok_s // NWORK
    blk = min(chunk, ROWS_PER_DMA)
    nblk = chunk // blk
    mesh = plsc.VectorSubcoreMesh(core_axis_name="c", subcore_axis_name="s")
    body = functools.partial(_sc_gather_kernel, off=off, chunk=chunk)
    fn = functools.partial(
        pl.kernel,
        mesh=mesh,
        out_type=jax.ShapeDtypeStruct((tok_s, EMB), jnp.float32),
        scratch_types=[
            pltpu.VMEM((chunk + 8,), jnp.int32),
            pltpu.VMEM((nblk, blk), jnp.int32),
            pltpu.VMEM((chunk, EMB), jnp.float32),
            pltpu.SemaphoreType.DMA,
        ],
    )(body)
    return fn(ids_flat, table)


def _mm_slice_body(x_ref, w_ref, *rest, off, tok_s):
    o_hbm, ob, sems = rest[-3], rest[-2], rest[-1]
    nc = tok_s // CT
    w = w_ref[...].astype(jnp.bfloat16)
    cps = [None] * NBUF
    for j in range(nc):
        b = j % NBUF
        if cps[b] is not None:
            cps[b].wait()
        xj = x_ref[pl.ds(j * CT, CT), :].astype(jnp.bfloat16)
        ob[b] = lax.dot_general(
            xj, w,
            dimension_numbers=(((1,), (1,)), ((), ())),
            preferred_element_type=jnp.float32,
        )
        cps[b] = pltpu.make_async_copy(
            ob.at[b], o_hbm.at[pl.ds(off + j * CT, CT), :], sems.at[b]
        )
        cps[b].start()
    for b in range(NBUF):
        if cps[b] is not None:
            cps[b].wait()


def _project_slice(emb_s, proj_w, out, off, tok_s):
    body = functools.partial(_mm_slice_body, off=off, tok_s=tok_s)
    vmem = pl.BlockSpec(memory_space=pltpu.MemorySpace.VMEM)
    hbm = pl.BlockSpec(memory_space=pltpu.MemorySpace.HBM)
    scratch = [
        pltpu.VMEM((NBUF, CT, DM), jnp.float32),
        pltpu.SemaphoreType.DMA((NBUF,)),
    ]
    if out is None:
        return pl.pallas_call(
            body,
            in_specs=[vmem, vmem],
            out_specs=hbm,
            out_shape=jax.ShapeDtypeStruct((NTOK, DM), jnp.float32),
            scratch_shapes=scratch,
        )(emb_s, proj_w)
    return pl.pallas_call(
        body,
        in_specs=[vmem, vmem, hbm],
        out_specs=hbm,
        out_shape=jax.ShapeDtypeStruct((NTOK, DM), jnp.float32),
        scratch_shapes=scratch,
        input_output_aliases={2: 0},
    )(emb_s, proj_w, out)


@jax.jit
def kernel(input_ids, bigram_emb, proj_w):
    ids_flat = input_ids.reshape(-1)
    offs = []
    o = 0
    for t in SLICES:
        offs.append(o)
        o += t
    embs = [
        _sc_gather(ids_flat, bigram_emb, off, tok_s)
        for off, tok_s in zip(offs, SLICES)
    ]
    out = None
    for emb_s, off, tok_s in zip(embs, offs, SLICES):
        out = _project_slice(emb_s, proj_w, out, off, tok_s)
    return out.reshape(BATCH, SEQ, DM)
